# baseline (speedup 1.0000x reference)
import functools

import jax
import jax.numpy as jnp
from jax import lax
from jax.experimental import pallas as pl
from jax.experimental.pallas import tpu as pltpu
from jax.experimental.pallas import tpu_sc as plsc

EMB = 32
SCALE = float(EMB) ** 0.5

NC = 2
NS = 16
NW = NC * NS

CHUNK = 128
NBUF = 2


def _body(xt_hbm, tq_hbm, out_hbm, idx_v, q_v, lb_v, rowbufs, outbufs, isem,
          gsems, wsems, *, npos, rows_per_w):
    wid = lax.axis_index("s") * NC + lax.axis_index("c")
    i_base = wid * rows_per_w
    nchunk = rows_per_w // CHUNK

    idx_cps = [
        pltpu.async_copy(
            xt_hbm.at[j, pl.ds(i_base, rows_per_w)], idx_v.at[j], isem)
        for j in range(npos)
    ]
    for cp in idx_cps:
        cp.wait()

    @pl.loop(0, npos)
    def _prep(j):
        for k in range(rows_per_w // 16):
            v = idx_v[j, pl.ds(k * 16, 16)]
            q_v[j, pl.ds(k * 16, 16)] = lax.shift_right_logical(v, 2)
            lb_v[j, pl.ds(k * 16, 16)] = lax.shift_left(
                jnp.bitwise_and(v, 3), 5)

    def fire_gather(j, c, b):
        return pltpu.async_copy(
            tq_hbm.at[q_v.at[j, pl.ds(c * CHUNK, CHUNK)]],
            rowbufs[b], gsems[b])

    def drain_gather(b):
        pltpu.make_async_copy(
            tq_hbm.at[pl.ds(0, CHUNK)], rowbufs[b], gsems[b]).wait()

    def drain_write(b):
        pltpu.make_async_copy(
            outbufs[b],
            out_hbm.at[0, pl.ds(0, EMB), pl.ds(0, CHUNK)], wsems[b],
        ).wait()

    iota = lax.iota(jnp.int32, 16)
    row_sel = [ii0 * 16 + iota for ii0 in range(8)]

    fire_gather(0, 0, 0)
    fire_gather(0, 1, 1)

    @pl.loop(0, npos)
    def _per_pos(j):
        for c in range(nchunk):
            b = c % NBUF
            drain_gather(b)
            if c < NBUF:
                @pl.when(j > 0)
                def _():
                    drain_write(b)
            else:
                drain_write(b)

            src = rowbufs[b]
            dst = outbufs[b]
            lbs = [lb_v[j, pl.ds(c * CHUNK + ii0 * 16, 16)]
                   for ii0 in range(8)]

            @pl.loop(0, EMB)
            def _tr(f):
                for ii0 in range(8):
                    v = plsc.load_gather(src, [row_sel[ii0], lbs[ii0] + f])
                    dst[f, pl.ds(ii0 * 16, 16)] = v * SCALE

            if c + NBUF < nchunk:
                fire_gather(j, c + NBUF, b)
            else:
                @pl.when(j + 1 < npos)
                def _():
                    fire_gather(j + 1, c + NBUF - nchunk, b)
            pltpu.async_copy(
                dst,
                out_hbm.at[j, pl.ds(0, EMB),
                           pl.ds(i_base + c * CHUNK, CHUNK)],
                wsems[b])

    for b in range(NBUF):
        drain_write(b)


@jax.jit
def _embed(xt, tq):
    npos, nbatch = xt.shape
    rows_per_w = nbatch // NW
    mesh = plsc.VectorSubcoreMesh(core_axis_name="c", subcore_axis_name="s")

    def body(xt_hbm, tq_hbm, out_hbm, idx_v, q_v, lb_v, *rest):
        rowbufs = rest[:NBUF]
        outbufs = rest[NBUF:2 * NBUF]
        isem = rest[2 * NBUF]
        gsems = rest[2 * NBUF + 1:2 * NBUF + 1 + NBUF]
        wsems = rest[2 * NBUF + 1 + NBUF:]
        _body(xt_hbm, tq_hbm, out_hbm, idx_v, q_v, lb_v, rowbufs, outbufs,
              isem, gsems, wsems, npos=npos, rows_per_w=rows_per_w)

    k = pl.kernel(
        body,
        out_type=jax.ShapeDtypeStruct((npos, EMB, nbatch), jnp.float32),
        mesh=mesh,
        scratch_types=(
            [pltpu.VMEM((npos, rows_per_w), jnp.int32) for _ in range(3)]
            + [pltpu.VMEM((CHUNK, 128), jnp.float32) for _ in range(NBUF)]
            + [pltpu.VMEM((EMB, CHUNK), jnp.float32) for _ in range(NBUF)]
            + [pltpu.SemaphoreType.DMA for _ in range(1 + 2 * NBUF)]
        ),
        compiler_params=pltpu.CompilerParams(
            use_tc_tiling_on_sc=True, needs_layout_passes=False),
    )
    return k(xt, tq)


def kernel(x, table):
    vocab, emb = table.shape
    xt = x.T.astype(jnp.int32)
    tq = table.reshape(vocab // 4, 4 * emb)
    out = _embed(xt, tq)
    return out.transpose(2, 0, 1)


# speedup vs baseline: 1.0038x; 1.0038x over previous
import functools

import jax
import jax.numpy as jnp
from jax import lax
from jax.experimental import pallas as pl
from jax.experimental.pallas import tpu as pltpu
from jax.experimental.pallas import tpu_sc as plsc

EMB = 32
SCALE = float(EMB) ** 0.5

NC = 2
NS = 16
NW = NC * NS

CHUNK = 128
NBUF = 4


def _body(xt_hbm, tq_hbm, out_hbm, idx_v, q_v, lb_v, rowbufs, outbufs, isem,
          gsems, wsems, *, npos, rows_per_w):
    wid = lax.axis_index("s") * NC + lax.axis_index("c")
    i_base = wid * rows_per_w
    nchunk = rows_per_w // CHUNK

    idx_cps = [
        pltpu.async_copy(
            xt_hbm.at[j, pl.ds(i_base, rows_per_w)], idx_v.at[j], isem)
        for j in range(npos)
    ]
    for cp in idx_cps:
        cp.wait()

    @pl.loop(0, npos)
    def _prep(j):
        for k in range(rows_per_w // 16):
            v = idx_v[j, pl.ds(k * 16, 16)]
            q_v[j, pl.ds(k * 16, 16)] = lax.shift_right_logical(v, 2)
            lb_v[j, pl.ds(k * 16, 16)] = lax.shift_left(
                jnp.bitwise_and(v, 3), 5)

    def fire_gather(j, c, b):
        return pltpu.async_copy(
            tq_hbm.at[q_v.at[j, pl.ds(c * CHUNK, CHUNK)]],
            rowbufs[b], gsems[b])

    def drain_gather(b):
        pltpu.make_async_copy(
            tq_hbm.at[pl.ds(0, CHUNK)], rowbufs[b], gsems[b]).wait()

    def drain_write(b):
        pltpu.make_async_copy(
            outbufs[b],
            out_hbm.at[0, pl.ds(0, EMB), pl.ds(0, CHUNK)], wsems[b],
        ).wait()

    iota = lax.iota(jnp.int32, 16)
    row_sel = [ii0 * 16 + iota for ii0 in range(8)]

    for c in range(nchunk):
        fire_gather(0, c, c)

    @pl.loop(0, npos)
    def _per_pos(j):
        for c in range(nchunk):
            b = c
            drain_gather(b)

            @pl.when(j > 0)
            def _():
                drain_write(b)

            src = rowbufs[b]
            dst = outbufs[b]
            lbs = [lb_v[j, pl.ds(c * CHUNK + ii0 * 16, 16)]
                   for ii0 in range(8)]

            @pl.loop(0, EMB)
            def _tr(f):
                for ii0 in range(8):
                    v = plsc.load_gather(src, [row_sel[ii0], lbs[ii0] + f])
                    dst[f, pl.ds(ii0 * 16, 16)] = v * SCALE

            @pl.when(j + 1 < npos)
            def _():
                fire_gather(j + 1, c, b)
            pltpu.async_copy(
                dst,
                out_hbm.at[j, pl.ds(0, EMB),
                           pl.ds(i_base + c * CHUNK, CHUNK)],
                wsems[b])

    for b in range(NBUF):
        drain_write(b)


@jax.jit
def _embed(xt, tq):
    npos, nbatch = xt.shape
    rows_per_w = nbatch // NW
    mesh = plsc.VectorSubcoreMesh(core_axis_name="c", subcore_axis_name="s")

    def body(xt_hbm, tq_hbm, out_hbm, idx_v, q_v, lb_v, *rest):
        rowbufs = rest[:NBUF]
        outbufs = rest[NBUF:2 * NBUF]
        isem = rest[2 * NBUF]
        gsems = rest[2 * NBUF + 1:2 * NBUF + 1 + NBUF]
        wsems = rest[2 * NBUF + 1 + NBUF:]
        _body(xt_hbm, tq_hbm, out_hbm, idx_v, q_v, lb_v, rowbufs, outbufs,
              isem, gsems, wsems, npos=npos, rows_per_w=rows_per_w)

    k = pl.kernel(
        body,
        out_type=jax.ShapeDtypeStruct((npos, EMB, nbatch), jnp.float32),
        mesh=mesh,
        scratch_types=(
            [pltpu.VMEM((npos, rows_per_w), jnp.int32) for _ in range(3)]
            + [pltpu.VMEM((CHUNK, 128), jnp.float32) for _ in range(NBUF)]
            + [pltpu.VMEM((EMB, CHUNK), jnp.float32) for _ in range(NBUF)]
            + [pltpu.SemaphoreType.DMA for _ in range(1 + 2 * NBUF)]
        ),
        compiler_params=pltpu.CompilerParams(
            use_tc_tiling_on_sc=True, needs_layout_passes=False),
    )
    return k(xt, tq)


def kernel(x, table):
    vocab, emb = table.shape
    xt = x.T.astype(jnp.int32)
    tq = table.reshape(vocab // 4, 4 * emb)
    out = _embed(xt, tq)
    return out.transpose(2, 0, 1)
